# baseline (speedup 1.0000x reference)
import functools

import jax
import jax.numpy as jnp
from jax import lax
from jax.experimental import pallas as pl
from jax.experimental.pallas import tpu as pltpu
from jax.experimental.pallas import tpu_sc as plsc

EMB = 64
CH = 3


def _scs_body(table_hbm, w_hbm, b_hbm, idx_hbm, out_hbm,
              t_s, w_s, b_s, idx_s, out_s, sem):
    c0 = pltpu.async_copy(idx_hbm, idx_s, sem)
    c1 = pltpu.async_copy(table_hbm, t_s, sem)
    c2 = pltpu.async_copy(w_hbm, w_s, sem)
    c3 = pltpu.async_copy(b_hbm, b_s, sem)
    c0.wait()
    c1.wait()
    c2.wait()
    c3.wait()

    i = idx_s[0]

    def step(g, acc):
        a0, a1, a2 = acc
        base = g * 4
        for u in range(4):
            d = base + u
            t = t_s[i, d]
            a0 = a0 + t * w_s[0, d]
            a1 = a1 + t * w_s[1, d]
            a2 = a2 + t * w_s[2, d]
        return (a0, a1, a2)

    acc = lax.fori_loop(0, EMB // 4, step, (b_s[0], b_s[1], b_s[2]))
    for c in range(CH):
        out_s[c] = acc[c]
    pltpu.sync_copy(out_s, out_hbm)


@jax.jit
def _run(table, W, b, idx):
    mesh = plsc.ScalarSubcoreMesh(axis_name="c", num_cores=1)
    f = functools.partial(
        pl.kernel,
        mesh=mesh,
        compiler_params=pltpu.CompilerParams(needs_layout_passes=False),
        out_type=jax.ShapeDtypeStruct((CH,), jnp.float32),
        scratch_types=[
            pltpu.SMEM((CH, EMB), jnp.float32),
            pltpu.SMEM((CH, EMB), jnp.float32),
            pltpu.SMEM((CH,), jnp.float32),
            pltpu.SMEM((1,), jnp.int32),
            pltpu.SMEM((CH,), jnp.float32),
            pltpu.SemaphoreType.DMA,
        ],
    )(_scs_body)
    return f(table, W, b, idx)


def kernel(table, W, b, idx):
    return _run(table, W, b, idx)


# speedup vs baseline: 1.0052x; 1.0052x over previous
import functools

import jax
import jax.numpy as jnp
from jax import lax
from jax.experimental import pallas as pl
from jax.experimental.pallas import tpu as pltpu
from jax.experimental.pallas import tpu_sc as plsc

EMB = 64
CH = 3


def _sc_body(table_hbm, w_hbm, b_hbm, idx_hbm, out_hbm,
             t_s, w_s, b_s, idx_s, out_s, sem):
    c0 = pltpu.async_copy(idx_hbm, idx_s, sem)
    c1 = pltpu.async_copy(table_hbm, t_s, sem)
    c2 = pltpu.async_copy(w_hbm, w_s, sem)
    c3 = pltpu.async_copy(b_hbm, b_s, sem)
    c0.wait()
    c1.wait()
    c2.wait()
    c3.wait()

    i = idx_s[0]

    def step(d, acc):
        t = t_s[i, d]
        return (acc[0] + t * w_s[0, d],
                acc[1] + t * w_s[1, d],
                acc[2] + t * w_s[2, d])

    acc = lax.fori_loop(0, EMB, step, (b_s[0], b_s[1], b_s[2]))
    for c in range(CH):
        out_s[c] = acc[c]
    pltpu.sync_copy(out_s, out_hbm)


@jax.jit
def _run(table, W, b, idx):
    mesh = plsc.ScalarSubcoreMesh(axis_name="c", num_cores=1)
    f = functools.partial(
        pl.kernel,
        mesh=mesh,
        compiler_params=pltpu.CompilerParams(needs_layout_passes=False),
        out_type=jax.ShapeDtypeStruct((CH,), jnp.float32),
        scratch_types=[
            pltpu.SMEM((CH, EMB), jnp.float32),
            pltpu.SMEM((CH, EMB), jnp.float32),
            pltpu.SMEM((CH,), jnp.float32),
            pltpu.SMEM((1,), jnp.int32),
            pltpu.SMEM((CH,), jnp.float32),
            pltpu.SemaphoreType.DMA,
        ],
    )(_sc_body)
    return f(table, W, b, idx)


def kernel(table, W, b, idx):
    return _run(table, W, b, idx)
